# baseline (speedup 1.0000x reference)
import math

import numpy as np
import jax
import jax.numpy as jnp
from jax import lax
from jax.experimental import pallas as pl
from jax.experimental.pallas import tpu as pltpu
from jax.experimental.pallas import tpu_sc as plsc

_NUM_EMB = 1000000
_DIM = 64
_BATCH = 4096
_SEQ = 200
_LANES = 16

_NC, _NS = 2, 16
_NW = _NC * _NS
_BBLK = _BATCH // _NW
_NG = _BBLK // _LANES


def _pos_encoding():
    pos = np.arange(_SEQ, dtype=np.float32)[:, None]
    div = np.exp(np.arange(0, _DIM, 2, dtype=np.float32)
                 * -(math.log(10000.0) / _DIM))
    pe = np.zeros((_SEQ, _DIM), dtype=np.float32)
    pe[:, 0::2] = np.sin(pos * div)
    pe[:, 1::2] = np.cos(pos * div)
    return pe.reshape(-1)


_PE = _pos_encoding()


def _body(xt_hbm, pe_hbm, table_hbm, out_hbm,
          xall_v, pe_v, gath0, gath1, ost0, ost1,
          gsm0, gsm1, osm0, osm1):
    wid = lax.axis_index("s") * _NC + lax.axis_index("c")
    b0 = wid * _BBLK
    pltpu.sync_copy(pe_hbm, pe_v)
    pltpu.sync_copy(xt_hbm.at[:, pl.ds(b0, _BBLK)], xall_v)
    iota = lax.iota(jnp.int32, _LANES)
    rowids = [g * _LANES + iota for g in range(_NG)]
    gaths = (gath0, gath1)
    osts = (ost0, ost1)
    gsms = (gsm0, gsm1)
    osms = (osm0, osm1)

    def start_gather(l, b):
        pltpu.async_copy(table_hbm.at[xall_v.at[l]], gaths[b], gsms[b])

    start_gather(0, 0)

    def step2(j, carry):
        for b in range(2):
            l = 2 * j + b

            @pl.when(l + 1 < _SEQ)
            def _():
                start_gather(l + 1, 1 - b)

            pltpu.make_async_copy(table_hbm.at[xall_v.at[l]], gaths[b],
                                  gsms[b]).wait()

            @pl.when(l >= 2)
            def _():
                pltpu.make_async_copy(
                    osts[b], out_hbm.at[0, :, pl.ds(b0, _BBLK)],
                    osms[b]).wait()

            pe_base = iota * 0 + l * _DIM

            @plsc.parallel_loop(0, _DIM, unroll=16)
            def col(d):
                pv = plsc.load_gather(pe_v, [pe_base + d])
                cd = iota * 0 + d
                for g in range(_NG):
                    cv = plsc.load_gather(gaths[b], [rowids[g], cd])
                    osts[b][d, pl.ds(g * _LANES, _LANES)] = cv + pv

            pltpu.async_copy(osts[b], out_hbm.at[l, :, pl.ds(b0, _BBLK)],
                             osms[b])
        return carry

    lax.fori_loop(0, _SEQ // 2, step2, 0)
    for b in range(2):
        pltpu.make_async_copy(osts[b], out_hbm.at[0, :, pl.ds(b0, _BBLK)],
                              osms[b]).wait()


@jax.jit
def _run(xt, pe, table):
    mesh = plsc.VectorSubcoreMesh(core_axis_name="c", subcore_axis_name="s")
    f = pl.kernel(
        _body,
        out_type=jax.ShapeDtypeStruct((_SEQ, _DIM, _BATCH), jnp.float32),
        mesh=mesh,
        scratch_types=[
            pltpu.VMEM((_SEQ, _BBLK), jnp.int32),
            pltpu.VMEM((_SEQ * _DIM,), jnp.float32),
            pltpu.VMEM((_BBLK, _DIM), jnp.float32),
            pltpu.VMEM((_BBLK, _DIM), jnp.float32),
            pltpu.VMEM((_DIM, _BBLK), jnp.float32),
            pltpu.VMEM((_DIM, _BBLK), jnp.float32),
            pltpu.SemaphoreType.DMA,
            pltpu.SemaphoreType.DMA,
            pltpu.SemaphoreType.DMA,
            pltpu.SemaphoreType.DMA,
        ],
        compiler_params=pltpu.CompilerParams(use_tc_tiling_on_sc=False,
                                             needs_layout_passes=False),
    )
    return f(xt, pe, table)


def kernel(x, table):
    out2 = _run(x.T, _PE, table)
    return jnp.transpose(out2, (2, 0, 1))


# speedup vs baseline: 1.2056x; 1.2056x over previous
import math

import numpy as np
import jax
import jax.numpy as jnp
from jax import lax
from jax.experimental import pallas as pl
from jax.experimental.pallas import tpu as pltpu
from jax.experimental.pallas import tpu_sc as plsc

_NUM_EMB = 1000000
_DIM = 64
_BATCH = 4096
_SEQ = 200
_LANES = 16

_NC, _NS = 2, 16
_NW = _NC * _NS
_SEQ_PER_W = _BATCH // _NW
_CHUNK_SEQ = 2
_ROWS = _CHUNK_SEQ * _SEQ
_N_CHUNKS = _SEQ_PER_W // _CHUNK_SEQ


def _pos_encoding():
    pos = np.arange(_SEQ, dtype=np.float32)[:, None]
    div = np.exp(np.arange(0, _DIM, 2, dtype=np.float32)
                 * -(math.log(10000.0) / _DIM))
    pe = np.zeros((_SEQ, _DIM), dtype=np.float32)
    pe[:, 0::2] = np.sin(pos * div)
    pe[:, 1::2] = np.cos(pos * div)
    return np.tile(pe, (_CHUNK_SEQ, 1))


_PE = _pos_encoding()


def _body(x_hbm, pe_hbm, table_hbm, out_hbm,
          idx0, idx1, rows0, rows1, pe_v, gsm0, gsm1, osm0, osm1):
    wid = lax.axis_index("s") * _NC + lax.axis_index("c")
    seq0 = wid * _SEQ_PER_W
    pltpu.sync_copy(pe_hbm, pe_v)
    idxs = (idx0, idx1)
    rows = (rows0, rows1)
    gsms = (gsm0, gsm1)
    osms = (osm0, osm1)

    def start_gather(i, b):
        row0 = (seq0 + i * _CHUNK_SEQ) * _SEQ
        pltpu.sync_copy(x_hbm.at[pl.ds(row0, _ROWS)], idxs[b])
        pltpu.async_copy(table_hbm.at[idxs[b]], rows[b], gsms[b])

    start_gather(0, 0)

    def step2(j, carry):
        for b in range(2):
            i = 2 * j + b
            s0 = seq0 + i * _CHUNK_SEQ

            @pl.when(i + 1 < _N_CHUNKS)
            def _():
                start_gather(i + 1, 1 - b)

            pltpu.make_async_copy(table_hbm.at[idxs[b]], rows[b],
                                  gsms[b]).wait()

            @pl.when(i >= 2)
            def _():
                for h in range(_CHUNK_SEQ):
                    pltpu.make_async_copy(
                        rows[b].at[pl.ds(h * _SEQ, _SEQ)],
                        out_hbm.at[s0 + h], osms[b]).wait()

            @plsc.parallel_loop(0, _ROWS, unroll=16)
            def add_row(r):
                for q in range(_DIM // _LANES):
                    sl = pl.ds(q * _LANES, _LANES)
                    rows[b][r, sl] = rows[b][r, sl] + pe_v[r, sl]

            for h in range(_CHUNK_SEQ):
                pltpu.async_copy(rows[b].at[pl.ds(h * _SEQ, _SEQ)],
                                 out_hbm.at[s0 + h], osms[b])
        return carry

    lax.fori_loop(0, _N_CHUNKS // 2, step2, 0)
    for b in range(2):
        for h in range(_CHUNK_SEQ):
            pltpu.make_async_copy(rows[b].at[pl.ds(h * _SEQ, _SEQ)],
                                  out_hbm.at[h], osms[b]).wait()


@jax.jit
def _run(x_flat, pe, table):
    mesh = plsc.VectorSubcoreMesh(core_axis_name="c", subcore_axis_name="s")
    f = pl.kernel(
        _body,
        out_type=jax.ShapeDtypeStruct((_BATCH, _SEQ, _DIM), jnp.float32),
        mesh=mesh,
        scratch_types=[
            pltpu.VMEM((_ROWS,), jnp.int32),
            pltpu.VMEM((_ROWS,), jnp.int32),
            pltpu.VMEM((_ROWS, _DIM), jnp.float32),
            pltpu.VMEM((_ROWS, _DIM), jnp.float32),
            pltpu.VMEM((_ROWS, _DIM), jnp.float32),
            pltpu.SemaphoreType.DMA,
            pltpu.SemaphoreType.DMA,
            pltpu.SemaphoreType.DMA,
            pltpu.SemaphoreType.DMA,
        ],
        compiler_params=pltpu.CompilerParams(use_tc_tiling_on_sc=False),
    )
    return f(x_flat, pe, table)


def kernel(x, table):
    return _run(x.reshape(-1), _PE, table)
